# baseline (speedup 1.0000x reference)
import jax
import jax.numpy as jnp
from jax import lax
from jax.experimental import pallas as pl
from jax.experimental.pallas import tpu as pltpu
from jax.experimental.pallas import tpu_sc as plsc

N_ROWS = 100000
N_SEG = 64
D = 128
NC = 2
NS = 16
NW = NC * NS
RPW = N_ROWS // NW
CH = 125
NCH = RPW // CH
NB = 7
NV = 196
BPAD = 3152
SENT = 127
MAGIC = 0x5F3759DF


def _sc_body(x_hbm, b_hbm, y_hbm, ps_hbm, pc_hbm,
             xb0, xb1, xb2, xb3, xb4, xb5, xb6, bvm, norms, ends_r, ends_i,
             cm_r, cm_i, out_s, out_c, insem, outsem, bsem, ysem):
    xbufs = (xb0, xb1, xb2, xb3, xb4, xb5, xb6)
    cid = lax.axis_index("c")
    sid = lax.axis_index("s")
    w = sid * NC + cid
    base = w * RPW
    base_al = pl.multiple_of(
        jnp.minimum(base - lax.rem(base, 8), N_ROWS - 3136), 8)
    shift = base - base_al

    bcp = pltpu.async_copy(b_hbm.at[pl.ds(base_al, 3136)],
                           bvm.at[pl.ds(0, 3136)], bsem)
    for b in range(NB):
        off = pl.multiple_of((base + b * CH) * D, 8)
        pltpu.async_copy(x_hbm.at[pl.ds(off, CH * D)], xbufs[b],
                         insem.at[b])
    bcp.wait()

    iota16 = lax.iota(jnp.int32, 16)
    zero16 = jnp.zeros((16,), jnp.float32)
    lane15 = iota16 == 15
    plsc.store_scatter(bvm, [iota16 + (shift + RPW)],
                       jnp.full((16,), SENT, jnp.int32))
    norms[pl.ds(RPW - 5, 16)] = zero16

    for c in range(NCH):
        b = c % NB
        off0 = pl.multiple_of((base + c * CH) * D, 8)
        pltpu.make_async_copy(x_hbm.at[pl.ds(off0, CH * D)], xbufs[b],
                              insem.at[b]).wait()
        ocp = pltpu.async_copy(xbufs[b], y_hbm.at[pl.ds(off0, CH * D)],
                               outsem.at[b])

        @plsc.parallel_loop(0, CH, 1, unroll=8)
        def _rows(r, _c=c, _xb=xbufs[b]):
            acc = zero16
            rb = r * D
            for k in range(8):
                v = _xb[pl.ds(rb + k * 16, 16)]
                acc = acc + v * v
            cs = plsc.cumsum(acc)
            plsc.store_scatter(
                norms, [jnp.full((16,), _c * CH + r, jnp.int32)],
                cs, mask=lane15)

        ocp.wait()
        if c + NB < NCH:
            offn = pl.multiple_of((base + (c + NB) * CH) * D, 8)
            pltpu.async_copy(x_hbm.at[pl.ds(offn, CH * D)],
                             xbufs[b], insem.at[b])

    @plsc.parallel_loop(0, NV, 1, unroll=4)
    def _newton(j):
        off = j * 16
        x = norms[pl.ds(off, 16)]
        y = plsc.bitcast(jnp.int32(MAGIC) - lax.shift_right_logical(
            plsc.bitcast(x, jnp.int32), 1), jnp.float32)
        for _i in range(3):
            y = y * (1.5 - 0.5 * x * y * y)
        norms[pl.ds(off, 16)] = x * y

    zi16 = jnp.zeros((16,), jnp.int32)
    for j in range(8):
        ends_r[pl.ds(j * 16, 16)] = zero16
        ends_i[pl.ds(j * 16, 16)] = zi16

    def vbody(j, carry, _iota=iota16):
        offb = shift + j * 16
        b16 = plsc.load_gather(bvm, [_iota + offb])
        bn16 = plsc.load_gather(bvm, [_iota + (offb + 1)])
        n16 = norms[pl.ds(j * 16, 16)]
        cs = plsc.cumsum(n16) + carry
        m = b16 != bn16
        plsc.store_scatter(ends_r, [b16], cs, mask=m)
        plsc.store_scatter(ends_i, [b16], _iota + (j * 16 + 1), mask=m)
        return carry + jnp.sum(n16)

    lax.fori_loop(0, NV, vbody, jnp.float32(0.0))

    cm_r[pl.ds(0, 16)] = zero16
    cm_i[pl.ds(0, 16)] = zi16
    c_r = jnp.float32(0.0)
    c_i = jnp.int32(0)
    for j in range(4):
        dst = iota16 + (j * 16 + 1)
        vr = jnp.maximum(plsc.cummax(ends_r[pl.ds(j * 16, 16)]), c_r)
        plsc.store_scatter(cm_r, [dst], vr)
        out_s[pl.ds(j * 16, 16)] = vr - cm_r[pl.ds(j * 16, 16)]
        c_r = jnp.max(vr)
        vi = jnp.maximum(plsc.cummax(ends_i[pl.ds(j * 16, 16)]), c_i)
        plsc.store_scatter(cm_i, [dst], vi)
        out_c[pl.ds(j * 16, 16)] = vi - cm_i[pl.ds(j * 16, 16)]
        c_i = jnp.max(vi)

    pw = pl.multiple_of(w * N_SEG, 8)
    pltpu.sync_copy(out_s, ps_hbm.at[pl.ds(pw, N_SEG)])
    pltpu.sync_copy(out_c, pc_hbm.at[pl.ds(pw, N_SEG)])


_sc_call = pl.kernel(
    _sc_body,
    out_type=[
        jax.ShapeDtypeStruct((N_ROWS * D,), jnp.float32),
        jax.ShapeDtypeStruct((NW * N_SEG,), jnp.float32),
        jax.ShapeDtypeStruct((NW * N_SEG,), jnp.int32),
    ],
    mesh=plsc.VectorSubcoreMesh(core_axis_name="c", subcore_axis_name="s"),
    compiler_params=pltpu.CompilerParams(needs_layout_passes=False),
    scratch_types=[
        pltpu.VMEM((CH * D,), jnp.float32),
        pltpu.VMEM((CH * D,), jnp.float32),
        pltpu.VMEM((CH * D,), jnp.float32),
        pltpu.VMEM((CH * D,), jnp.float32),
        pltpu.VMEM((CH * D,), jnp.float32),
        pltpu.VMEM((CH * D,), jnp.float32),
        pltpu.VMEM((CH * D,), jnp.float32),
        pltpu.VMEM((BPAD,), jnp.int32),
        pltpu.VMEM((3136,), jnp.float32),
        pltpu.VMEM((128,), jnp.float32),
        pltpu.VMEM((128,), jnp.int32),
        pltpu.VMEM((80,), jnp.float32),
        pltpu.VMEM((80,), jnp.int32),
        pltpu.VMEM((64,), jnp.float32),
        pltpu.VMEM((64,), jnp.int32),
        pltpu.SemaphoreType.DMA((NB,)),
        pltpu.SemaphoreType.DMA((NB,)),
        pltpu.SemaphoreType.DMA,
        pltpu.SemaphoreType.DMA,
    ],
)


def _combine_body(ps_ref, pc_ref, b_ref, o_ref):
    sums = jnp.sum(ps_ref[...], axis=0, keepdims=True)
    cnts = jnp.sum(pc_ref[...].astype(jnp.float32), axis=0, keepdims=True)
    bs = b_ref[0, 0, 999]
    wtd = sums / cnts
    mask = jax.lax.broadcasted_iota(jnp.int32, (1, N_SEG), 1) < bs
    nm = jnp.sum(jnp.where(mask, wtd, 0.0)) / (bs + 1).astype(jnp.float32)
    o_ref[...] = nm.reshape(1, 1)


def kernel(input, batch):
    y, ps, pc = _sc_call(input.reshape(N_ROWS * D), batch)
    y = y.reshape(N_ROWS, D)
    ps = ps.reshape(NW, N_SEG)
    pc = pc.reshape(NW, N_SEG)
    batch3 = batch.reshape(100, 1, 1000)
    nm = pl.pallas_call(
        _combine_body,
        grid=(1,),
        in_specs=[
            pl.BlockSpec((NW, N_SEG), lambda i: (0, 0)),
            pl.BlockSpec((NW, N_SEG), lambda i: (0, 0)),
            pl.BlockSpec((1, 1, 1000), lambda i: (99, 0, 0)),
        ],
        out_specs=pl.BlockSpec((1, 1), lambda i: (0, 0)),
        out_shape=jax.ShapeDtypeStruct((1, 1), jnp.float32),
    )(ps, pc, batch3)
    return y, nm.reshape(())


# speedup vs baseline: 1.0803x; 1.0803x over previous
import jax
import jax.numpy as jnp
from jax import lax
from jax.experimental import pallas as pl
from jax.experimental.pallas import tpu as pltpu
from jax.experimental.pallas import tpu_sc as plsc

N_ROWS = 100000
N_SEG = 64
D = 128
NC = 2
NS = 16
NW = NC * NS
RPW = N_ROWS // NW
CH = 250
CHUNKS = [(i * CH, CH) for i in range(12)] + [(3000, 125)]
NCH = len(CHUNKS)
NB = 3
NV = 196
BPAD = 3152
SENT = 127
MAGIC = 0x5F3759DF


def _sc_body(x_hbm, b_hbm, y_hbm, ps_hbm, pc_hbm,
             xb0, xb1, xb2, bvm, norms, ends_r, ends_i,
             cm_r, cm_i, out_s, out_c, insem, outsem, bsem, ysem):
    xbufs = (xb0, xb1, xb2)
    cid = lax.axis_index("c")
    sid = lax.axis_index("s")
    w = sid * NC + cid
    base = w * RPW
    base_al = pl.multiple_of(
        jnp.minimum(base - lax.rem(base, 8), N_ROWS - 3136), 8)
    shift = base - base_al

    bcp = pltpu.async_copy(b_hbm.at[pl.ds(base_al, 3136)],
                           bvm.at[pl.ds(0, 3136)], bsem)
    for b in range(NB):
        r0, nr = CHUNKS[b]
        off = pl.multiple_of((base + r0) * D, 8)
        pltpu.async_copy(x_hbm.at[pl.ds(off, nr * D)],
                         xbufs[b].at[pl.ds(0, nr * D)], insem.at[b])
    bcp.wait()

    iota16 = lax.iota(jnp.int32, 16)
    zero16 = jnp.zeros((16,), jnp.float32)
    lane15 = iota16 == 15
    plsc.store_scatter(bvm, [iota16 + (shift + RPW)],
                       jnp.full((16,), SENT, jnp.int32))
    norms[pl.ds(RPW - 5, 16)] = zero16

    for c in range(NCH):
        b = c % NB
        r0, nr = CHUNKS[c]
        off0 = pl.multiple_of((base + r0) * D, 8)
        pltpu.make_async_copy(x_hbm.at[pl.ds(off0, nr * D)],
                              xbufs[b].at[pl.ds(0, nr * D)],
                              insem.at[b]).wait()
        ocp = pltpu.async_copy(xbufs[b].at[pl.ds(0, nr * D)],
                               y_hbm.at[pl.ds(off0, nr * D)], outsem.at[b])

        @plsc.parallel_loop(0, nr, 1, unroll=4)
        def _rows(r, _r0=r0, _xb=xbufs[b]):
            acc = zero16
            rb = r * D
            for k in range(8):
                v = _xb[pl.ds(rb + k * 16, 16)]
                acc = acc + v * v
            cs = plsc.cumsum(acc)
            plsc.store_scatter(
                norms, [jnp.full((16,), _r0 + r, jnp.int32)],
                cs, mask=lane15)

        ocp.wait()
        if c + NB < NCH:
            rn, nn = CHUNKS[c + NB]
            offn = pl.multiple_of((base + rn) * D, 8)
            pltpu.async_copy(x_hbm.at[pl.ds(offn, nn * D)],
                             xbufs[b].at[pl.ds(0, nn * D)], insem.at[b])

    @plsc.parallel_loop(0, NV, 1, unroll=4)
    def _newton(j):
        off = j * 16
        x = norms[pl.ds(off, 16)]
        y = plsc.bitcast(jnp.int32(MAGIC) - lax.shift_right_logical(
            plsc.bitcast(x, jnp.int32), 1), jnp.float32)
        for _i in range(3):
            y = y * (1.5 - 0.5 * x * y * y)
        norms[pl.ds(off, 16)] = x * y

    zi16 = jnp.zeros((16,), jnp.int32)
    for j in range(8):
        ends_r[pl.ds(j * 16, 16)] = zero16
        ends_i[pl.ds(j * 16, 16)] = zi16

    def vbody(j, carry, _iota=iota16):
        offb = shift + j * 16
        b16 = plsc.load_gather(bvm, [_iota + offb])
        bn16 = plsc.load_gather(bvm, [_iota + (offb + 1)])
        n16 = norms[pl.ds(j * 16, 16)]
        cs = plsc.cumsum(n16) + carry
        m = b16 != bn16
        plsc.store_scatter(ends_r, [b16], cs, mask=m)
        plsc.store_scatter(ends_i, [b16], _iota + (j * 16 + 1), mask=m)
        return carry + jnp.sum(n16)

    lax.fori_loop(0, NV, vbody, jnp.float32(0.0))

    cm_r[pl.ds(0, 16)] = zero16
    cm_i[pl.ds(0, 16)] = zi16
    c_r = jnp.float32(0.0)
    c_i = jnp.int32(0)
    for j in range(4):
        dst = iota16 + (j * 16 + 1)
        vr = jnp.maximum(plsc.cummax(ends_r[pl.ds(j * 16, 16)]), c_r)
        plsc.store_scatter(cm_r, [dst], vr)
        out_s[pl.ds(j * 16, 16)] = vr - cm_r[pl.ds(j * 16, 16)]
        c_r = jnp.max(vr)
        vi = jnp.maximum(plsc.cummax(ends_i[pl.ds(j * 16, 16)]), c_i)
        plsc.store_scatter(cm_i, [dst], vi)
        out_c[pl.ds(j * 16, 16)] = vi - cm_i[pl.ds(j * 16, 16)]
        c_i = jnp.max(vi)

    pw = pl.multiple_of(w * N_SEG, 8)
    pltpu.sync_copy(out_s, ps_hbm.at[pl.ds(pw, N_SEG)])
    pltpu.sync_copy(out_c, pc_hbm.at[pl.ds(pw, N_SEG)])


_sc_call = pl.kernel(
    _sc_body,
    out_type=[
        jax.ShapeDtypeStruct((N_ROWS * D,), jnp.float32),
        jax.ShapeDtypeStruct((NW * N_SEG,), jnp.float32),
        jax.ShapeDtypeStruct((NW * N_SEG,), jnp.int32),
    ],
    mesh=plsc.VectorSubcoreMesh(core_axis_name="c", subcore_axis_name="s"),
    compiler_params=pltpu.CompilerParams(needs_layout_passes=False),
    scratch_types=[
        pltpu.VMEM((CH * D,), jnp.float32),
        pltpu.VMEM((CH * D,), jnp.float32),
        pltpu.VMEM((CH * D,), jnp.float32),
        pltpu.VMEM((BPAD,), jnp.int32),
        pltpu.VMEM((3136,), jnp.float32),
        pltpu.VMEM((128,), jnp.float32),
        pltpu.VMEM((128,), jnp.int32),
        pltpu.VMEM((80,), jnp.float32),
        pltpu.VMEM((80,), jnp.int32),
        pltpu.VMEM((64,), jnp.float32),
        pltpu.VMEM((64,), jnp.int32),
        pltpu.SemaphoreType.DMA((NB,)),
        pltpu.SemaphoreType.DMA((NB,)),
        pltpu.SemaphoreType.DMA,
        pltpu.SemaphoreType.DMA,
    ],
)


def _combine_body(ps_ref, pc_ref, b_ref, o_ref):
    sums = jnp.sum(ps_ref[...], axis=0, keepdims=True)
    cnts = jnp.sum(pc_ref[...].astype(jnp.float32), axis=0, keepdims=True)
    bs = b_ref[0, 0, 999]
    wtd = sums / cnts
    mask = jax.lax.broadcasted_iota(jnp.int32, (1, N_SEG), 1) < bs
    nm = jnp.sum(jnp.where(mask, wtd, 0.0)) / (bs + 1).astype(jnp.float32)
    o_ref[...] = nm.reshape(1, 1)


def kernel(input, batch):
    y, ps, pc = _sc_call(input.reshape(N_ROWS * D), batch)
    y = y.reshape(N_ROWS, D)
    ps = ps.reshape(NW, N_SEG)
    pc = pc.reshape(NW, N_SEG)
    batch3 = batch.reshape(100, 1, 1000)
    nm = pl.pallas_call(
        _combine_body,
        grid=(1,),
        in_specs=[
            pl.BlockSpec((NW, N_SEG), lambda i: (0, 0)),
            pl.BlockSpec((NW, N_SEG), lambda i: (0, 0)),
            pl.BlockSpec((1, 1, 1000), lambda i: (99, 0, 0)),
        ],
        out_specs=pl.BlockSpec((1, 1), lambda i: (0, 0)),
        out_shape=jax.ShapeDtypeStruct((1, 1), jnp.float32),
    )(ps, pc, batch3)
    return y, nm.reshape(())


# speedup vs baseline: 1.1016x; 1.0198x over previous
import jax
import jax.numpy as jnp
from jax import lax
from jax.experimental import pallas as pl
from jax.experimental.pallas import tpu as pltpu
from jax.experimental.pallas import tpu_sc as plsc

N_ROWS = 100000
N_SEG = 64
D = 128
NC = 2
NS = 16
NW = NC * NS
RPW = N_ROWS // NW
CH = 312
CHUNKS = [(i * CH, CH) for i in range(10)] + [(3120, 5)]
NCH = len(CHUNKS)
NB = 3
NV = 196
BPAD = 3152
SENT = 127
MAGIC = 0x5F3759DF


def _sc_body(x_hbm, b_hbm, y_hbm, ps_hbm, pc_hbm,
             xb0, xb1, xb2, bvm, norms, ends_r, ends_i,
             cm_r, cm_i, out_s, out_c, insem, outsem, bsem, ysem):
    xbufs = (xb0, xb1, xb2)
    cid = lax.axis_index("c")
    sid = lax.axis_index("s")
    w = sid * NC + cid
    base = w * RPW
    base_al = pl.multiple_of(
        jnp.minimum(base - lax.rem(base, 8), N_ROWS - 3136), 8)
    shift = base - base_al

    bcp = pltpu.async_copy(b_hbm.at[pl.ds(base_al, 3136)],
                           bvm.at[pl.ds(0, 3136)], bsem)
    for b in range(NB):
        r0, nr = CHUNKS[b]
        off = pl.multiple_of((base + r0) * D, 8)
        pltpu.async_copy(x_hbm.at[pl.ds(off, nr * D)],
                         xbufs[b].at[pl.ds(0, nr * D)], insem.at[b])
    bcp.wait()

    iota16 = lax.iota(jnp.int32, 16)
    zero16 = jnp.zeros((16,), jnp.float32)
    lane15 = iota16 == 15
    plsc.store_scatter(bvm, [iota16 + (shift + RPW)],
                       jnp.full((16,), SENT, jnp.int32))
    norms[pl.ds(RPW - 5, 16)] = zero16

    for c in range(NCH):
        b = c % NB
        r0, nr = CHUNKS[c]
        off0 = pl.multiple_of((base + r0) * D, 8)
        pltpu.make_async_copy(x_hbm.at[pl.ds(off0, nr * D)],
                              xbufs[b].at[pl.ds(0, nr * D)],
                              insem.at[b]).wait()
        ocp = pltpu.async_copy(xbufs[b].at[pl.ds(0, nr * D)],
                               y_hbm.at[pl.ds(off0, nr * D)], outsem.at[b])

        @plsc.parallel_loop(0, nr, 1, unroll=4)
        def _rows(r, _r0=r0, _xb=xbufs[b]):
            acc = zero16
            rb = r * D
            for k in range(8):
                v = _xb[pl.ds(rb + k * 16, 16)]
                acc = acc + v * v
            cs = plsc.cumsum(acc)
            plsc.store_scatter(
                norms, [jnp.full((16,), _r0 + r, jnp.int32)],
                cs, mask=lane15)

        ocp.wait()
        if c + NB < NCH:
            rn, nn = CHUNKS[c + NB]
            offn = pl.multiple_of((base + rn) * D, 8)
            pltpu.async_copy(x_hbm.at[pl.ds(offn, nn * D)],
                             xbufs[b].at[pl.ds(0, nn * D)], insem.at[b])

    @plsc.parallel_loop(0, NV, 1, unroll=4)
    def _newton(j):
        off = j * 16
        x = norms[pl.ds(off, 16)]
        y = plsc.bitcast(jnp.int32(MAGIC) - lax.shift_right_logical(
            plsc.bitcast(x, jnp.int32), 1), jnp.float32)
        for _i in range(3):
            y = y * (1.5 - 0.5 * x * y * y)
        norms[pl.ds(off, 16)] = x * y

    zi16 = jnp.zeros((16,), jnp.int32)
    for j in range(8):
        ends_r[pl.ds(j * 16, 16)] = zero16
        ends_i[pl.ds(j * 16, 16)] = zi16

    def vbody(j, carry, _iota=iota16):
        offb = shift + j * 16
        b16 = plsc.load_gather(bvm, [_iota + offb])
        bn16 = plsc.load_gather(bvm, [_iota + (offb + 1)])
        n16 = norms[pl.ds(j * 16, 16)]
        cs = plsc.cumsum(n16) + carry
        m = b16 != bn16
        plsc.store_scatter(ends_r, [b16], cs, mask=m)
        plsc.store_scatter(ends_i, [b16], _iota + (j * 16 + 1), mask=m)
        return carry + jnp.sum(n16)

    lax.fori_loop(0, NV, vbody, jnp.float32(0.0))

    cm_r[pl.ds(0, 16)] = zero16
    cm_i[pl.ds(0, 16)] = zi16
    c_r = jnp.float32(0.0)
    c_i = jnp.int32(0)
    for j in range(4):
        dst = iota16 + (j * 16 + 1)
        vr = jnp.maximum(plsc.cummax(ends_r[pl.ds(j * 16, 16)]), c_r)
        plsc.store_scatter(cm_r, [dst], vr)
        out_s[pl.ds(j * 16, 16)] = vr - cm_r[pl.ds(j * 16, 16)]
        c_r = jnp.max(vr)
        vi = jnp.maximum(plsc.cummax(ends_i[pl.ds(j * 16, 16)]), c_i)
        plsc.store_scatter(cm_i, [dst], vi)
        out_c[pl.ds(j * 16, 16)] = vi - cm_i[pl.ds(j * 16, 16)]
        c_i = jnp.max(vi)

    pw = pl.multiple_of(w * N_SEG, 8)
    pltpu.sync_copy(out_s, ps_hbm.at[pl.ds(pw, N_SEG)])
    pltpu.sync_copy(out_c, pc_hbm.at[pl.ds(pw, N_SEG)])


_sc_call = pl.kernel(
    _sc_body,
    out_type=[
        jax.ShapeDtypeStruct((N_ROWS * D,), jnp.float32),
        jax.ShapeDtypeStruct((NW * N_SEG,), jnp.float32),
        jax.ShapeDtypeStruct((NW * N_SEG,), jnp.int32),
    ],
    mesh=plsc.VectorSubcoreMesh(core_axis_name="c", subcore_axis_name="s"),
    compiler_params=pltpu.CompilerParams(needs_layout_passes=False),
    scratch_types=[
        pltpu.VMEM((CH * D,), jnp.float32),
        pltpu.VMEM((CH * D,), jnp.float32),
        pltpu.VMEM((CH * D,), jnp.float32),
        pltpu.VMEM((BPAD,), jnp.int32),
        pltpu.VMEM((3136,), jnp.float32),
        pltpu.VMEM((128,), jnp.float32),
        pltpu.VMEM((128,), jnp.int32),
        pltpu.VMEM((80,), jnp.float32),
        pltpu.VMEM((80,), jnp.int32),
        pltpu.VMEM((64,), jnp.float32),
        pltpu.VMEM((64,), jnp.int32),
        pltpu.SemaphoreType.DMA((NB,)),
        pltpu.SemaphoreType.DMA((NB,)),
        pltpu.SemaphoreType.DMA,
        pltpu.SemaphoreType.DMA,
    ],
)


def _combine_body(ps_ref, pc_ref, b_ref, o_ref):
    sums = jnp.sum(ps_ref[...], axis=0, keepdims=True)
    cnts = jnp.sum(pc_ref[...].astype(jnp.float32), axis=0, keepdims=True)
    bs = b_ref[0, 0, 999]
    wtd = sums / cnts
    mask = jax.lax.broadcasted_iota(jnp.int32, (1, N_SEG), 1) < bs
    nm = jnp.sum(jnp.where(mask, wtd, 0.0)) / (bs + 1).astype(jnp.float32)
    o_ref[...] = nm.reshape(1, 1)


def kernel(input, batch):
    y, ps, pc = _sc_call(input.reshape(N_ROWS * D), batch)
    y = y.reshape(N_ROWS, D)
    ps = ps.reshape(NW, N_SEG)
    pc = pc.reshape(NW, N_SEG)
    batch3 = batch.reshape(100, 1, 1000)
    nm = pl.pallas_call(
        _combine_body,
        grid=(1,),
        in_specs=[
            pl.BlockSpec((NW, N_SEG), lambda i: (0, 0)),
            pl.BlockSpec((NW, N_SEG), lambda i: (0, 0)),
            pl.BlockSpec((1, 1, 1000), lambda i: (99, 0, 0)),
        ],
        out_specs=pl.BlockSpec((1, 1), lambda i: (0, 0)),
        out_shape=jax.ShapeDtypeStruct((1, 1), jnp.float32),
    )(ps, pc, batch3)
    return y, nm.reshape(())
